# baseline (speedup 1.0000x reference)
import functools

import jax
import jax.numpy as jnp
from jax import lax
from jax.experimental import pallas as pl
from jax.experimental.pallas import tpu as pltpu
from jax.experimental.pallas import tpu_sc as plsc

V, D, H = 1_000_000, 64, 100
L, B = 200, 16384

NC, NS = 2, 16
NW = NC * NS
BPW = B // NW
CL = 40
NCH = L // CL
FL = 20
NGR = CL // FL
DEPTH = 4

_mesh = plsc.VectorSubcoreMesh(core_axis_name="c", subcore_axis_name="s")


@functools.partial(
    pl.kernel,
    out_type=jax.ShapeDtypeStruct((B, D), jnp.bfloat16),
    mesh=_mesh,
    scratch_types=[
        pltpu.VMEM((2, CL, BPW), jnp.int32),
        pltpu.VMEM((BPW, D), jnp.bfloat16),
        pltpu.VMEM((BPW, D), jnp.float32),
        pltpu.SemaphoreType.DMA,
        pltpu.SemaphoreType.DMA,
    ],
    compiler_params=pltpu.CompilerParams(
        use_tc_tiling_on_sc=False, needs_layout_passes=False
    ),
)
def _sc_pool(inp_hbm, emb_hbm, out_hbm, idx_v, acc16, accf, sem, sem_i):
    wid = lax.axis_index("s") * NC + lax.axis_index("c")
    base = wid * BPW

    def stage_chunk(ci, buf):
        for r in range(CL):
            pltpu.async_copy(
                inp_hbm.at[pl.ds((ci * CL + r) * B + base, BPW)],
                idx_v.at[buf, r],
                sem_i,
            )

    def wait_chunk(buf):
        for r in range(CL):
            pltpu.make_async_copy(
                inp_hbm.at[pl.ds(base, BPW)], idx_v.at[buf, r], sem_i
            ).wait()

    zf = jnp.zeros((16,), jnp.float32)
    zb = jnp.zeros((32,), jnp.bfloat16)

    def zero_body(i, carry):
        for c in range(D // 16):
            accf[i, pl.ds(c * 16, 16)] = zf
        for c in range(D // 32):
            acc16[i, pl.ds(c * 32, 32)] = zb
        return carry

    lax.fori_loop(0, BPW, zero_body, 0)

    def fire(buf, l):
        pltpu.async_copy(
            emb_hbm.at[idx_v.at[buf, l]], acc16, sem, add=True
        )

    def drain_one(buf, l):
        pltpu.make_async_copy(
            emb_hbm.at[idx_v.at[buf, l]], acc16, sem
        ).wait()

    def flush_body(i, carry):
        for c in range(D // 32):
            a, b = plsc.unpack(
                acc16[i, pl.ds(c * 32, 32)], format=plsc.PackFormat.INTERLEAVED
            )
            accf[i, pl.ds(c * 32, 16)] = accf[i, pl.ds(c * 32, 16)] + a
            accf[i, pl.ds(c * 32 + 16, 16)] = (
                accf[i, pl.ds(c * 32 + 16, 16)] + b
            )
            acc16[i, pl.ds(c * 32, 32)] = zb
        return carry

    stage_chunk(0, 0)
    wait_chunk(0)
    for ci in range(NCH):
        buf = ci % 2
        if ci + 1 < NCH:
            stage_chunk(ci + 1, 1 - buf)

        for gi in range(NGR):
            g0 = gi * FL
            for l in range(g0, g0 + DEPTH):
                fire(buf, l)

            def l_body(l, carry):
                fire(buf, l)
                drain_one(buf, l - DEPTH)
                return carry

            lax.fori_loop(g0 + DEPTH, g0 + FL, l_body, 0)

            for l in range(g0 + FL - DEPTH, g0 + FL):
                drain_one(buf, l)

            lax.fori_loop(0, BPW, flush_body, 0)

        if ci + 1 < NCH:
            wait_chunk(1 - buf)

    def emit_body(i, carry):
        for c in range(D // 32):
            acc16[i, pl.ds(c * 32, 32)] = plsc.pack(
                accf[i, pl.ds(c * 32, 16)],
                accf[i, pl.ds(c * 32 + 16, 16)],
                format=plsc.PackFormat.INTERLEAVED,
            )
        return carry

    lax.fori_loop(0, BPW, emit_body, 0)
    pltpu.sync_copy(acc16, out_hbm.at[pl.ds(base, BPW)])


def _mlp_body(x_ref, w1_ref, b1_ref, w2_ref, b2_ref, o_ref):
    x = x_ref[...].astype(jnp.float32) * (1.0 / L)
    h = jnp.maximum(
        jnp.dot(x, w1_ref[...], preferred_element_type=jnp.float32)
        + b1_ref[...],
        0.0,
    )
    o_ref[...] = (
        jnp.dot(h, w2_ref[...], preferred_element_type=jnp.float32)
        + b2_ref[...]
    )


_BM = 2048


def _mlp(pooled_sum, W1, b1, W2, b2):
    return pl.pallas_call(
        _mlp_body,
        grid=(B // _BM,),
        in_specs=[
            pl.BlockSpec((_BM, D), lambda i: (i, 0)),
            pl.BlockSpec((D, H), lambda i: (0, 0)),
            pl.BlockSpec((1, H), lambda i: (0, 0)),
            pl.BlockSpec((H, 1), lambda i: (0, 0)),
            pl.BlockSpec((1, 1), lambda i: (0, 0)),
        ],
        out_specs=pl.BlockSpec((_BM, 1), lambda i: (i, 0)),
        out_shape=jax.ShapeDtypeStruct((B, 1), jnp.float32),
    )(pooled_sum, W1, b1.reshape(1, H), W2, b2.reshape(1, 1))


def kernel(input, emb, W1, b1, W2, b2):
    pooled_sum = _sc_pool(input.reshape(-1), emb.astype(jnp.bfloat16))
    out = _mlp(pooled_sum, W1, b1, W2, b2)
    return out[:, :, None]


# speedup vs baseline: 1.1597x; 1.1597x over previous
import functools

import jax
import jax.numpy as jnp
from jax import lax
from jax.experimental import pallas as pl
from jax.experimental.pallas import tpu as pltpu
from jax.experimental.pallas import tpu_sc as plsc

V, D, H = 1_000_000, 64, 100
L, B = 200, 16384

NC, NS = 2, 16
NW = NC * NS
BPW = B // NW
GCH = 512
NG = BPW // GCH
CL = 40
NCH = L // CL

_mesh = plsc.VectorSubcoreMesh(core_axis_name="c", subcore_axis_name="s")


@functools.partial(
    pl.kernel,
    out_type=jax.ShapeDtypeStruct((B, D), jnp.float32),
    mesh=_mesh,
    scratch_types=[
        pltpu.VMEM((2, CL, BPW), jnp.int32),
        pltpu.VMEM((BPW, D), jnp.float32),
        pltpu.SemaphoreType.DMA,
        pltpu.SemaphoreType.DMA,
    ],
    compiler_params=pltpu.CompilerParams(use_tc_tiling_on_sc=False),
)
def _sc_pool(inp_hbm, emb_hbm, out_hbm, idx_v, acc_v, sem, sem_i):
    wid = lax.axis_index("s") * NC + lax.axis_index("c")
    base = wid * BPW

    def stage_chunk(ci, buf):
        for r in range(CL):
            pltpu.async_copy(
                inp_hbm.at[pl.ds((ci * CL + r) * B + base, BPW)],
                idx_v.at[buf, r],
                sem_i,
            )

    def wait_chunk(buf):
        for r in range(CL):
            pltpu.make_async_copy(
                inp_hbm.at[pl.ds(base, BPW)], idx_v.at[buf, r], sem_i
            ).wait()

    zeros16 = jnp.zeros((16,), jnp.float32)

    def zero_body(i, carry):
        for c in range(D // 16):
            acc_v[i, pl.ds(c * 16, 16)] = zeros16
        return carry

    lax.fori_loop(0, BPW, zero_body, 0)

    DEPTH = 8

    def fire(buf, l):
        for g in range(NG):
            pltpu.async_copy(
                emb_hbm.at[idx_v.at[buf, l, pl.ds(g * GCH, GCH)]],
                acc_v.at[pl.ds(g * GCH, GCH), :],
                sem,
                add=True,
            )

    def drain_one(buf, l):
        for g in range(NG):
            pltpu.make_async_copy(
                emb_hbm.at[idx_v.at[buf, l, pl.ds(g * GCH, GCH)]],
                acc_v.at[pl.ds(g * GCH, GCH), :],
                sem,
            ).wait()

    stage_chunk(0, 0)
    wait_chunk(0)
    for ci in range(NCH):
        buf = ci % 2
        if ci + 1 < NCH:
            stage_chunk(ci + 1, 1 - buf)

        for l in range(DEPTH):
            fire(buf, l)

        def l_body(l, carry):
            fire(buf, l)
            drain_one(buf, l - DEPTH)
            return carry

        lax.fori_loop(DEPTH, CL, l_body, 0)

        for l in range(CL - DEPTH, CL):
            drain_one(buf, l)

        if ci + 1 < NCH:
            wait_chunk(1 - buf)

    pltpu.sync_copy(acc_v, out_hbm.at[pl.ds(base, BPW)])


def _mlp_body(x_ref, w1_ref, b1_ref, w2_ref, b2_ref, o_ref):
    x = x_ref[...] * (1.0 / L)
    h = jnp.maximum(
        jnp.dot(x, w1_ref[...], preferred_element_type=jnp.float32)
        + b1_ref[...],
        0.0,
    )
    o_ref[...] = (
        jnp.dot(h, w2_ref[...], preferred_element_type=jnp.float32)
        + b2_ref[...]
    )


_BM = 2048


def _mlp(pooled_sum, W1, b1, W2, b2):
    return pl.pallas_call(
        _mlp_body,
        grid=(B // _BM,),
        in_specs=[
            pl.BlockSpec((_BM, D), lambda i: (i, 0)),
            pl.BlockSpec((D, H), lambda i: (0, 0)),
            pl.BlockSpec((1, H), lambda i: (0, 0)),
            pl.BlockSpec((H, 1), lambda i: (0, 0)),
            pl.BlockSpec((1, 1), lambda i: (0, 0)),
        ],
        out_specs=pl.BlockSpec((_BM, 1), lambda i: (i, 0)),
        out_shape=jax.ShapeDtypeStruct((B, 1), jnp.float32),
    )(pooled_sum, W1, b1.reshape(1, H), W2, b2.reshape(1, 1))


def kernel(input, emb, W1, b1, W2, b2):
    pooled_sum = _sc_pool(input.reshape(-1), emb)
    out = _mlp(pooled_sum, W1, b1, W2, b2)
    return out[:, :, None]


# speedup vs baseline: 1.1605x; 1.0007x over previous
import functools

import jax
import jax.numpy as jnp
from jax import lax
from jax.experimental import pallas as pl
from jax.experimental.pallas import tpu as pltpu
from jax.experimental.pallas import tpu_sc as plsc

V, D, H = 1_000_000, 64, 100
L, B = 200, 16384

NC, NS = 2, 16
NW = NC * NS
BPW = B // NW
GCH = 512
NG = BPW // GCH
CL = 40
NCH = L // CL

_mesh = plsc.VectorSubcoreMesh(core_axis_name="c", subcore_axis_name="s")


@functools.partial(
    pl.kernel,
    out_type=jax.ShapeDtypeStruct((B, D), jnp.float32),
    mesh=_mesh,
    scratch_types=[
        pltpu.VMEM((2, CL, BPW), jnp.int32),
        pltpu.VMEM((BPW, D), jnp.float32),
        pltpu.SemaphoreType.DMA,
        pltpu.SemaphoreType.DMA,
    ],
    compiler_params=pltpu.CompilerParams(use_tc_tiling_on_sc=False),
)
def _sc_pool(inp_hbm, emb_hbm, out_hbm, idx_v, acc_v, sem, sem_i):
    wid = lax.axis_index("s") * NC + lax.axis_index("c")
    base = wid * BPW

    def stage_chunk(ci, buf):
        for r in range(CL):
            pltpu.async_copy(
                inp_hbm.at[pl.ds((ci * CL + r) * B + base, BPW)],
                idx_v.at[buf, r],
                sem_i,
            )

    def wait_chunk(buf):
        for r in range(CL):
            pltpu.make_async_copy(
                inp_hbm.at[pl.ds(base, BPW)], idx_v.at[buf, r], sem_i
            ).wait()

    stage_chunk(0, 0)

    zeros16 = jnp.zeros((16,), jnp.float32)

    def zero_body(i, carry):
        for c in range(D // 16):
            acc_v[i, pl.ds(c * 16, 16)] = zeros16
        return carry

    lax.fori_loop(0, BPW, zero_body, 0)

    DEPTH = 8

    def fire(buf, l):
        for g in range(NG):
            pltpu.async_copy(
                emb_hbm.at[idx_v.at[buf, l, pl.ds(g * GCH, GCH)]],
                acc_v.at[pl.ds(g * GCH, GCH), :],
                sem,
                add=True,
            )

    def drain_one(buf, l):
        for g in range(NG):
            pltpu.make_async_copy(
                emb_hbm.at[idx_v.at[buf, l, pl.ds(g * GCH, GCH)]],
                acc_v.at[pl.ds(g * GCH, GCH), :],
                sem,
            ).wait()

    wait_chunk(0)
    for ci in range(NCH):
        buf = ci % 2
        if ci + 1 < NCH:
            stage_chunk(ci + 1, 1 - buf)

        for l in range(DEPTH):
            fire(buf, l)

        def l_body(l, carry):
            fire(buf, l)
            drain_one(buf, l - DEPTH)
            return carry

        lax.fori_loop(DEPTH, CL, l_body, 0)

        for l in range(CL - DEPTH, CL):
            drain_one(buf, l)

        if ci + 1 < NCH:
            wait_chunk(1 - buf)

    pltpu.sync_copy(acc_v, out_hbm.at[pl.ds(base, BPW)])


def _mlp_body(x_ref, w1_ref, b1_ref, w2_ref, b2_ref, o_ref):
    x = x_ref[...] * (1.0 / L)
    h = jnp.maximum(
        jnp.dot(x, w1_ref[...], preferred_element_type=jnp.float32)
        + b1_ref[...],
        0.0,
    )
    o_ref[...] = (
        jnp.dot(h, w2_ref[...], preferred_element_type=jnp.float32)
        + b2_ref[...]
    )


_BM = 2048


def _mlp(pooled_sum, W1, b1, W2, b2):
    return pl.pallas_call(
        _mlp_body,
        grid=(B // _BM,),
        in_specs=[
            pl.BlockSpec((_BM, D), lambda i: (i, 0)),
            pl.BlockSpec((D, H), lambda i: (0, 0)),
            pl.BlockSpec((1, H), lambda i: (0, 0)),
            pl.BlockSpec((H, 1), lambda i: (0, 0)),
            pl.BlockSpec((1, 1), lambda i: (0, 0)),
        ],
        out_specs=pl.BlockSpec((_BM, 1), lambda i: (i, 0)),
        out_shape=jax.ShapeDtypeStruct((B, 1), jnp.float32),
    )(pooled_sum, W1, b1.reshape(1, H), W2, b2.reshape(1, 1))


def kernel(input, emb, W1, b1, W2, b2):
    pooled_sum = _sc_pool(input.reshape(-1), emb)
    out = _mlp(pooled_sum, W1, b1, W2, b2)
    return out[:, :, None]
